# baseline (speedup 1.0000x reference)
import functools

import jax
import jax.numpy as jnp
from jax import lax
from jax.experimental import pallas as pl
from jax.experimental.pallas import tpu as pltpu
from jax.experimental.pallas import tpu_sc as plsc

B2 = 32768
D = 64
NW = 32
PER_W = B2 // NW
CHUNK = 512
IDX_W = 128
LMBDA = 0.01


def _sc_gather_score(h_idx, t_idx, r_idx, entity_emb, relation_emb):
    mesh = plsc.VectorSubcoreMesh(core_axis_name="c", subcore_axis_name="s")

    @functools.partial(
        pl.kernel,
        mesh=mesh,
        compiler_params=pltpu.CompilerParams(
            needs_layout_passes=False, use_tc_tiling_on_sc=False),
        out_type=[
            jax.ShapeDtypeStruct((B2,), jnp.float32),
            jax.ShapeDtypeStruct((NW, 16), jnp.float32),
        ],
        scratch_types=[
            pltpu.VMEM((CHUNK // IDX_W, IDX_W), jnp.int32),
            pltpu.VMEM((CHUNK // IDX_W, IDX_W), jnp.int32),
            pltpu.VMEM((CHUNK // IDX_W, IDX_W), jnp.int32),
            pltpu.VMEM((CHUNK, D), jnp.float32),
            pltpu.VMEM((CHUNK, D), jnp.float32),
            pltpu.VMEM((CHUNK, D), jnp.float32),
            pltpu.VMEM((CHUNK,), jnp.float32),
            pltpu.VMEM((16,), jnp.float32),
            pltpu.SemaphoreType.DMA,
        ],
    )
    def sc_kernel(hidx_hbm, tidx_hbm, ridx_hbm, ent_hbm, rel_hbm,
                  dots_hbm, sq_hbm,
                  hidx_v, tidx_v, ridx_v, h_rows, t_rows, r_rows,
                  dots_v, sq_v, sem):
        wid = lax.axis_index("s") * 2 + lax.axis_index("c")
        lane = lax.broadcasted_iota(jnp.int32, (16,), 0)

        def group_body(g, sq_acc):
            svec = jnp.zeros((16,), jnp.float32)
            for j in range(16):
                row = g * 16 + j
                acc = None
                for c in range(4):
                    hv = h_rows[row, pl.ds(c * 16, 16)]
                    tv = t_rows[row, pl.ds(c * 16, 16)]
                    rv = r_rows[row, pl.ds(c * 16, 16)]
                    p = hv * tv * rv
                    acc = p if acc is None else acc + p
                    sq_acc = sq_acc + (hv * hv + tv * tv + rv * rv)
                svec = jnp.where(lane == j, jnp.sum(acc), svec)
            dots_v[pl.ds(g * 16, 16)] = svec
            return sq_acc

        sq_acc = jnp.zeros((16,), jnp.float32)
        for chunk in range(PER_W // CHUNK):
            irow = wid * (PER_W // IDX_W) + chunk * (CHUNK // IDX_W)
            pltpu.sync_copy(hidx_hbm.at[pl.ds(irow, CHUNK // IDX_W)], hidx_v)
            pltpu.sync_copy(tidx_hbm.at[pl.ds(irow, CHUNK // IDX_W)], tidx_v)
            pltpu.sync_copy(ridx_hbm.at[pl.ds(irow, CHUNK // IDX_W)], ridx_v)
            descs = []
            for k in range(CHUNK // IDX_W):
                dst = pl.ds(k * IDX_W, IDX_W)
                descs.append(pltpu.async_copy(
                    ent_hbm.at[hidx_v.at[k]], h_rows.at[dst], sem))
                descs.append(pltpu.async_copy(
                    ent_hbm.at[tidx_v.at[k]], t_rows.at[dst], sem))
                descs.append(pltpu.async_copy(
                    rel_hbm.at[ridx_v.at[k]], r_rows.at[dst], sem))
            for dsc in descs:
                dsc.wait()
            sq_acc = lax.fori_loop(0, CHUNK // 16, group_body, sq_acc)
            pltpu.sync_copy(
                dots_v, dots_hbm.at[pl.ds(wid * PER_W + chunk * CHUNK, CHUNK)])
        sq_v[...] = sq_acc
        pltpu.sync_copy(sq_v, sq_hbm.at[wid])

    return sc_kernel(h_idx, t_idx, r_idx, entity_emb, relation_emb)


def _finalize(dots, sq):
    rows = B2 // 128

    def body(dots_ref, sq_ref, out_ref):
        s = dots_ref[...]
        rowid = lax.broadcasted_iota(jnp.int32, (rows, 128), 0)
        x = jnp.where(rowid < rows // 2, -s, s)
        sp = jnp.maximum(x, 0.0) + jnp.log1p(jnp.exp(-jnp.abs(x)))
        mean_sp = jnp.sum(sp) / float(B2)
        regul = jnp.sum(sq_ref[...]) / float(B2 * D)
        out_ref[...] = jnp.reshape(mean_sp + LMBDA * regul, (1, 1))

    return pl.pallas_call(
        body,
        out_shape=jax.ShapeDtypeStruct((1, 1), jnp.float32),
    )(dots.reshape(rows, 128), sq)


def kernel(pos_h, pos_r, pos_t, neg_h, neg_r, neg_t, entity_emb, relation_emb):
    h_idx = jnp.concatenate([pos_h, neg_h]).reshape(B2 // IDX_W, IDX_W)
    t_idx = jnp.concatenate([pos_t, neg_t]).reshape(B2 // IDX_W, IDX_W)
    r_idx = jnp.concatenate([pos_r[:, 0], neg_r[:, 0]]).reshape(B2 // IDX_W, IDX_W)
    dots, sq = _sc_gather_score(h_idx, t_idx, r_idx, entity_emb, relation_emb)
    return _finalize(dots, sq)[0, 0]
